# baseline (speedup 1.0000x reference)
import functools
from itertools import combinations

import jax
import jax.numpy as jnp
from jax import lax
from jax.experimental import pallas as pl
from jax.experimental.pallas import tpu as pltpu
from jax.experimental.pallas import tpu_sc as plsc

F = 26
V1 = 100001
E = 32
A = 64
B = 4096

_PAIRS = list(combinations(range(F), 2))
P = len(_PAIRS)
CI = [p[0] for p in _PAIRS]
CJ = [p[1] for p in _PAIRS]

NC = 2
NS = 16
NW = NC * NS
ROWS = B * F
RPW = ROWS // NW
NCHUNK_SC = RPW // 128

BB = 256
NB = B // BB
CP = 13
NCH = P // CP


def _sc_gather_body(table_hbm, idx_hbm, out_hbm, idx_v, rows_v, sem):
    wid = lax.axis_index("s") * NC + lax.axis_index("c")
    pltpu.sync_copy(idx_hbm.at[wid], idx_v)

    def half(h, carry):
        cps = []
        for j in range(NCHUNK_SC // 2):
            r = h * (NCHUNK_SC // 2) + j
            cps.append(pltpu.async_copy(table_hbm.at[idx_v.at[r]],
                                        rows_v.at[r], sem))
        for cp in cps:
            cp.wait()
        return carry

    lax.fori_loop(0, 2, half, 0)
    pltpu.sync_copy(rows_v, out_hbm.at[wid])


def _sc_gather(table_flat, idx):
    mesh = plsc.VectorSubcoreMesh(core_axis_name="c", subcore_axis_name="s")
    kern = pl.kernel(
        _sc_gather_body,
        out_type=jax.ShapeDtypeStruct((NW, NCHUNK_SC, 128, E), jnp.float32),
        mesh=mesh,
        scratch_types=[
            pltpu.VMEM((NCHUNK_SC, 128), jnp.int32),
            pltpu.VMEM((NCHUNK_SC, 128, E), jnp.float32),
            pltpu.SemaphoreType.DMA,
        ],
        compiler_params=pltpu.CompilerParams(use_tc_tiling_on_sc=False),
    )
    return kern(table_flat, idx)


def _afm_body(xt_ref, w1t_ref, b1_ref, w2_ref, fcw_ref, fcb_ref, out_ref,
              c_all, s_all):
    m = jnp.full((1, BB), -jnp.inf, jnp.float32)
    for c in range(NCH):
        pieces = []
        for k in range(CP):
            p = c * CP + k
            xi = xt_ref[CI[p] * E:(CI[p] + 1) * E, :]
            xj = xt_ref[CJ[p] * E:(CJ[p] + 1) * E, :]
            pieces.append(xi * xj)
        c_chunk = jnp.concatenate(pieces, axis=1)
        c_all[:, c * CP * BB:(c + 1) * CP * BB] = c_chunk
        h = jnp.dot(w1t_ref[:], c_chunk,
                    preferred_element_type=jnp.float32) + b1_ref[:]
        h = jnp.maximum(h, 0.0)
        s = jnp.sum(h * w2_ref[:], axis=0, keepdims=True)
        s_all[:, c * CP * BB:(c + 1) * CP * BB] = s
        for k in range(CP):
            m = jnp.maximum(m, s[:, k * BB:(k + 1) * BB])

    def body(p, carry):
        z, f = carry
        sp = s_all[:, pl.ds(p * BB, BB)]
        e = jnp.exp(sp - m)
        cp_blk = c_all[:, pl.ds(p * BB, BB)]
        return z + e, f + e * cp_blk

    z, f = lax.fori_loop(
        0, P, body,
        (jnp.zeros((1, BB), jnp.float32), jnp.zeros((E, BB), jnp.float32)))
    f = f / z
    y = jnp.sum(f * fcw_ref[:], axis=0, keepdims=True) + fcb_ref[:]
    out_ref[0] = 1.0 / (1.0 + jnp.exp(-y))


def _afm_tower(xt, w1t, b1c, w2, fcw, fcbc):
    grid = (NB,)
    return pl.pallas_call(
        _afm_body,
        grid=grid,
        in_specs=[
            pl.BlockSpec((F * E, BB), lambda b: (0, b)),
            pl.BlockSpec((A, E), lambda b: (0, 0)),
            pl.BlockSpec((A, 1), lambda b: (0, 0)),
            pl.BlockSpec((A, 1), lambda b: (0, 0)),
            pl.BlockSpec((E, 1), lambda b: (0, 0)),
            pl.BlockSpec((1, 1), lambda b: (0, 0)),
        ],
        out_specs=pl.BlockSpec((1, 1, BB), lambda b: (b, 0, 0)),
        out_shape=jax.ShapeDtypeStruct((NB, 1, BB), jnp.float32),
        scratch_shapes=[
            pltpu.VMEM((E, P * BB), jnp.float32),
            pltpu.VMEM((1, P * BB), jnp.float32),
        ],
    )(xt, w1t, b1c, w2, fcw, fcbc)


def kernel(f00, f01, f02, f03, f04, f05, f06, f07, f08, f09, f10, f11, f12,
           f13, f14, f15, f16, f17, f18, f19, f20, f21, f22, f23, f24, f25,
           tables, W1, b1, W2, fcW, fcb):
    feats = jnp.stack([f00, f01, f02, f03, f04, f05, f06, f07, f08, f09, f10,
                       f11, f12, f13, f14, f15, f16, f17, f18, f19, f20, f21,
                       f22, f23, f24, f25], axis=1)
    offs = (jnp.arange(F, dtype=feats.dtype) * V1)[None, :]
    idx = (feats + offs).astype(jnp.int32).reshape(NW, NCHUNK_SC, 128)

    table_flat = tables.reshape(F * V1, E)
    rows = _sc_gather(table_flat, idx)
    xt = rows.reshape(B, F * E).T

    out = _afm_tower(xt, W1.T, b1.reshape(A, 1), W2, fcW,
                     fcb.reshape(1, 1))
    return out.reshape(B)


# speedup vs baseline: 30.8489x; 30.8489x over previous
import functools
from itertools import combinations

import jax
import jax.numpy as jnp
from jax import lax
from jax.experimental import pallas as pl
from jax.experimental.pallas import tpu as pltpu
from jax.experimental.pallas import tpu_sc as plsc

F = 26
V1 = 100001
E = 32
A = 64
B = 4096

_PAIRS = list(combinations(range(F), 2))
P = len(_PAIRS)
CI = [p[0] for p in _PAIRS]
CJ = [p[1] for p in _PAIRS]

NC = 2
NS = 16
NW = NC * NS
HALF = F // 2

BB = 128
NB = B // BB
BG = B // 128
CP = 13
NCH = P // CP


VA = 50048
VB = V1 - VA


def _sc_gather_body(table_hbm, idx_hbm, out_hbm, slab_a, slab_b,
                    idx_a, idx_b, xrow_a, xrow_b, isem, ssem, wsem):
    w = lax.axis_index("s") * NC + lax.axis_index("c")
    idx_bufs = [idx_a, idx_b]
    xrow_bufs = [xrow_a, xrow_b]
    slabs = [slab_a, slab_b]

    def gather_half(buf, half):
        idx_ref = idx_bufs[buf]
        xrow_ref = xrow_bufs[buf]
        slab = slabs[half]

        def bstep(j, carry):
            iv = idx_ref[0, pl.ds(j * 16, 16)]
            if half == 0:
                msk = iv < VA
                off = jnp.minimum(iv, VA - 1)
            else:
                msk = iv >= VA
                off = jnp.maximum(iv - VA, 0)
            g = plsc.load_gather(slab, [off], mask=msk)
            old = xrow_ref[pl.ds(j * 16, 16)]
            xrow_ref[pl.ds(j * 16, 16)] = jnp.where(msk, g, old)
            return carry

        lax.fori_loop(0, B // 16, bstep, 0, unroll=4)

    i_cp = [None] * F
    sa_cp = [None] * F
    sb_cp = [None] * F
    w_cp = [None] * F

    def fire_idx(f):
        return pltpu.async_copy(idx_hbm.at[f], idx_bufs[f % 2], isem)

    def fire_slab(f, half):
        row = table_hbm.at[f * E + w]
        if half == 0:
            return pltpu.async_copy(row.at[pl.ds(0, VA)], slab_a, ssem)
        return pltpu.async_copy(row.at[pl.ds(VA, VB)], slab_b, ssem)

    i_cp[0] = fire_idx(0)
    sa_cp[0] = fire_slab(0, 0)
    sb_cp[0] = fire_slab(0, 1)
    for f in range(F):
        buf = f % 2
        if f >= 2:
            w_cp[f - 2].wait()
        i_cp[f].wait()
        if f + 1 < F:
            i_cp[f + 1] = fire_idx(f + 1)
        sa_cp[f].wait()
        gather_half(buf, 0)
        if f + 1 < F:
            sa_cp[f + 1] = fire_slab(f + 1, 0)
        sb_cp[f].wait()
        gather_half(buf, 1)
        if f + 1 < F:
            sb_cp[f + 1] = fire_slab(f + 1, 1)
        w_cp[f] = pltpu.async_copy(xrow_bufs[buf], out_hbm.at[f * E + w],
                                   wsem)
    w_cp[F - 2].wait()
    w_cp[F - 1].wait()


def _sc_gather(table2, idx3):
    mesh = plsc.VectorSubcoreMesh(core_axis_name="c", subcore_axis_name="s")
    kern = pl.kernel(
        _sc_gather_body,
        out_type=jax.ShapeDtypeStruct((F * E, B), jnp.float32),
        mesh=mesh,
        scratch_types=[
            pltpu.VMEM((VA,), jnp.float32),
            pltpu.VMEM((VB,), jnp.float32),
            pltpu.VMEM((1, B), jnp.int32),
            pltpu.VMEM((1, B), jnp.int32),
            pltpu.VMEM((B,), jnp.float32),
            pltpu.VMEM((B,), jnp.float32),
            pltpu.SemaphoreType.DMA,
            pltpu.SemaphoreType.DMA,
            pltpu.SemaphoreType.DMA,
        ],
        compiler_params=pltpu.CompilerParams(needs_layout_passes=False),
    )
    return kern(table2, idx3)


def _afm_body(xt_ref, w1t_ref, b1_ref, w2_ref, fcw_ref, fcb_ref, out_ref,
              c_all, s_all):
    xt = xt_ref[:]
    m = jnp.full((1, BB), -jnp.inf, jnp.float32)
    for c in range(NCH):
        pieces = []
        for k in range(CP):
            p = c * CP + k
            xi = xt[CI[p] * E:(CI[p] + 1) * E, :]
            xj = xt[CJ[p] * E:(CJ[p] + 1) * E, :]
            pieces.append(xi * xj)
        c_chunk = jnp.concatenate(pieces, axis=1)
        c_all[:, c * CP * BB:(c + 1) * CP * BB] = c_chunk
        h = jnp.dot(w1t_ref[:], c_chunk,
                    preferred_element_type=jnp.float32) + b1_ref[:]
        h = jnp.maximum(h, 0.0)
        s = jnp.sum(h * w2_ref[:], axis=0, keepdims=True)
        s_all[:, c * CP * BB:(c + 1) * CP * BB] = s
        for k in range(CP):
            m = jnp.maximum(m, s[:, k * BB:(k + 1) * BB])

    def body(p, carry):
        z, f = carry
        sp = s_all[:, pl.ds(p * BB, BB)]
        e = jnp.exp(sp - m)
        cp_blk = c_all[:, pl.ds(p * BB, BB)]
        return z + e, f + e * cp_blk

    z, f = lax.fori_loop(
        0, P, body,
        (jnp.zeros((1, BB), jnp.float32), jnp.zeros((E, BB), jnp.float32)))
    f = f / z
    y = jnp.sum(f * fcw_ref[:], axis=0, keepdims=True) + fcb_ref[:]
    out_ref[0] = 1.0 / (1.0 + jnp.exp(-y))


def _afm_tower(xt, w1t, b1c, w2, fcw, fcbc):
    return pl.pallas_call(
        _afm_body,
        grid=(NB,),
        in_specs=[
            pl.BlockSpec((F * E, BB), lambda b: (0, b)),
            pl.BlockSpec((A, E), lambda b: (0, 0)),
            pl.BlockSpec((A, 1), lambda b: (0, 0)),
            pl.BlockSpec((A, 1), lambda b: (0, 0)),
            pl.BlockSpec((E, 1), lambda b: (0, 0)),
            pl.BlockSpec((1, 1), lambda b: (0, 0)),
        ],
        out_specs=pl.BlockSpec((1, 1, BB), lambda b: (b, 0, 0)),
        out_shape=jax.ShapeDtypeStruct((NB, 1, BB), jnp.float32),
        scratch_shapes=[
            pltpu.VMEM((E, P * BB), jnp.float32),
            pltpu.VMEM((1, P * BB), jnp.float32),
        ],
    )(xt, w1t, b1c, w2, fcw, fcbc)


def kernel(f00, f01, f02, f03, f04, f05, f06, f07, f08, f09, f10, f11, f12,
           f13, f14, f15, f16, f17, f18, f19, f20, f21, f22, f23, f24, f25,
           tables, W1, b1, W2, fcW, fcb):
    featsT = jnp.stack([f00, f01, f02, f03, f04, f05, f06, f07, f08, f09, f10,
                        f11, f12, f13, f14, f15, f16, f17, f18, f19, f20, f21,
                        f22, f23, f24, f25], axis=0)
    idx3 = featsT.astype(jnp.int32).reshape(F, 1, B)

    table2 = jnp.swapaxes(tables, 1, 2).reshape(F * E, V1)
    xt = _sc_gather(table2, idx3)

    out = _afm_tower(xt, W1.T, b1.reshape(A, 1), W2, fcW,
                     fcb.reshape(1, 1))
    return out.reshape(B)


# speedup vs baseline: 42.3378x; 1.3724x over previous
import functools
from itertools import combinations

import jax
import jax.numpy as jnp
from jax import lax
from jax.experimental import pallas as pl
from jax.experimental.pallas import tpu as pltpu
from jax.experimental.pallas import tpu_sc as plsc

F = 26
V1 = 100001
E = 32
A = 64
B = 4096

_PAIRS = list(combinations(range(F), 2))
P = len(_PAIRS)
CI = [p[0] for p in _PAIRS]
CJ = [p[1] for p in _PAIRS]

NC = 2
NS = 16
NW = NC * NS
HALF = F // 2

BB = 256
NB = B // BB
BG = B // 128
CP = 13
NCH = P // CP


VA = 50048
VB = V1 - VA


def _sc_gather_body(table_hbm, idx_hbm, out_hbm, slab_a, slab_b,
                    idx_a, idx_b, xrow_a, xrow_b, isem, ssem, wsem):
    w = lax.axis_index("s") * NC + lax.axis_index("c")
    idx_bufs = [idx_a, idx_b]
    xrow_bufs = [xrow_a, xrow_b]
    slabs = [slab_a, slab_b]

    def gather_half(buf, half):
        idx_ref = idx_bufs[buf]
        xrow_ref = xrow_bufs[buf]
        slab = slabs[half]

        def bstep(j, carry):
            iv = idx_ref[0, pl.ds(j * 16, 16)]
            if half == 0:
                msk = iv < VA
                off = jnp.minimum(iv, VA - 1)
            else:
                msk = iv >= VA
                off = jnp.maximum(iv - VA, 0)
            g = plsc.load_gather(slab, [off], mask=msk)
            old = xrow_ref[pl.ds(j * 16, 16)]
            xrow_ref[pl.ds(j * 16, 16)] = jnp.where(msk, g, old)
            return carry

        lax.fori_loop(0, B // 16, bstep, 0, unroll=4)

    i_cp = [None] * F
    sa_cp = [None] * F
    sb_cp = [None] * F
    w_cp = [None] * F

    def fire_idx(f):
        return pltpu.async_copy(idx_hbm.at[f], idx_bufs[f % 2], isem)

    def fire_slab(f, half):
        row = table_hbm.at[f * E + w]
        if half == 0:
            return pltpu.async_copy(row.at[pl.ds(0, VA)], slab_a, ssem)
        return pltpu.async_copy(row.at[pl.ds(VA, VB)], slab_b, ssem)

    i_cp[0] = fire_idx(0)
    sa_cp[0] = fire_slab(0, 0)
    sb_cp[0] = fire_slab(0, 1)
    for f in range(F):
        buf = f % 2
        if f >= 2:
            w_cp[f - 2].wait()
        i_cp[f].wait()
        if f + 1 < F:
            i_cp[f + 1] = fire_idx(f + 1)
        sa_cp[f].wait()
        gather_half(buf, 0)
        if f + 1 < F:
            sa_cp[f + 1] = fire_slab(f + 1, 0)
        sb_cp[f].wait()
        gather_half(buf, 1)
        if f + 1 < F:
            sb_cp[f + 1] = fire_slab(f + 1, 1)
        w_cp[f] = pltpu.async_copy(xrow_bufs[buf], out_hbm.at[f * E + w],
                                   wsem)
    w_cp[F - 2].wait()
    w_cp[F - 1].wait()


def _sc_gather(table2, idx3):
    mesh = plsc.VectorSubcoreMesh(core_axis_name="c", subcore_axis_name="s")
    kern = pl.kernel(
        _sc_gather_body,
        out_type=jax.ShapeDtypeStruct((F * E, B), jnp.float32),
        mesh=mesh,
        scratch_types=[
            pltpu.VMEM((VA,), jnp.float32),
            pltpu.VMEM((VB,), jnp.float32),
            pltpu.VMEM((1, B), jnp.int32),
            pltpu.VMEM((1, B), jnp.int32),
            pltpu.VMEM((B,), jnp.float32),
            pltpu.VMEM((B,), jnp.float32),
            pltpu.SemaphoreType.DMA,
            pltpu.SemaphoreType.DMA,
            pltpu.SemaphoreType.DMA,
        ],
        compiler_params=pltpu.CompilerParams(needs_layout_passes=False),
    )
    return kern(table2, idx3)


def _afm_body(xt_ref, w1t_ref, b1_ref, w2_ref, fcw_ref, fcb_ref, out_ref,
              c_all, s_all):
    xt = xt_ref[:]
    m = jnp.full((1, BB), -jnp.inf, jnp.float32)
    for c in range(NCH):
        pieces = []
        for k in range(CP):
            p = c * CP + k
            xi = xt[CI[p] * E:(CI[p] + 1) * E, :]
            xj = xt[CJ[p] * E:(CJ[p] + 1) * E, :]
            pieces.append(xi * xj)
        c_chunk = jnp.concatenate(pieces, axis=1)
        c_all[:, c * CP * BB:(c + 1) * CP * BB] = c_chunk
        h = jnp.dot(w1t_ref[:], c_chunk,
                    preferred_element_type=jnp.float32) + b1_ref[:]
        h = jnp.maximum(h, 0.0)
        s = jnp.sum(h * w2_ref[:], axis=0, keepdims=True)
        s_all[:, c * CP * BB:(c + 1) * CP * BB] = s
        for k in range(CP):
            m = jnp.maximum(m, s[:, k * BB:(k + 1) * BB])

    def body(p, carry):
        z, f = carry
        sp = s_all[:, pl.ds(p * BB, BB)]
        e = jnp.exp(sp - m)
        cp_blk = c_all[:, pl.ds(p * BB, BB)]
        return z + e, f + e * cp_blk

    z, f = lax.fori_loop(
        0, P, body,
        (jnp.zeros((1, BB), jnp.float32), jnp.zeros((E, BB), jnp.float32)),
        unroll=5)
    f = f / z
    y = jnp.sum(f * fcw_ref[:], axis=0, keepdims=True) + fcb_ref[:]
    out_ref[0] = 1.0 / (1.0 + jnp.exp(-y))


def _afm_tower(xt, w1t, b1c, w2, fcw, fcbc):
    return pl.pallas_call(
        _afm_body,
        grid=(NB,),
        in_specs=[
            pl.BlockSpec((F * E, BB), lambda b: (0, b)),
            pl.BlockSpec((A, E), lambda b: (0, 0)),
            pl.BlockSpec((A, 1), lambda b: (0, 0)),
            pl.BlockSpec((A, 1), lambda b: (0, 0)),
            pl.BlockSpec((E, 1), lambda b: (0, 0)),
            pl.BlockSpec((1, 1), lambda b: (0, 0)),
        ],
        out_specs=pl.BlockSpec((1, 1, BB), lambda b: (b, 0, 0)),
        out_shape=jax.ShapeDtypeStruct((NB, 1, BB), jnp.float32),
        scratch_shapes=[
            pltpu.VMEM((E, P * BB), jnp.float32),
            pltpu.VMEM((1, P * BB), jnp.float32),
        ],
    )(xt, w1t, b1c, w2, fcw, fcbc)


def kernel(f00, f01, f02, f03, f04, f05, f06, f07, f08, f09, f10, f11, f12,
           f13, f14, f15, f16, f17, f18, f19, f20, f21, f22, f23, f24, f25,
           tables, W1, b1, W2, fcW, fcb):
    featsT = jnp.stack([f00, f01, f02, f03, f04, f05, f06, f07, f08, f09, f10,
                        f11, f12, f13, f14, f15, f16, f17, f18, f19, f20, f21,
                        f22, f23, f24, f25], axis=0)
    idx3 = featsT.astype(jnp.int32).reshape(F, 1, B)

    table2 = jnp.swapaxes(tables, 1, 2).reshape(F * E, V1)
    xt = _sc_gather(table2, idx3)

    out = _afm_tower(xt, W1.T, b1.reshape(A, 1), W2, fcW,
                     fcb.reshape(1, 1))
    return out.reshape(B)
